# Initial kernel scaffold; baseline (speedup 1.0000x reference)
import jax
import jax.numpy as jnp
from jax.experimental import pallas as pl


def kernel(time_vector, constant_offset, linear_trend, emd_seasonal_components, residual_amplitudes, residual_phases, residual_periods, emd_spatial_weights, local_spatial_weights, neighbor_indices, neighbor_weights, local_weights):
    raise NotImplementedError("write your pallas kernel here")


# speedup vs baseline: 1.2261x; 1.2261x over previous
import functools
import jax
import jax.numpy as jnp
import numpy as np
from jax import lax
from jax.experimental import pallas as pl
from jax.experimental.pallas import tpu as pltpu
from jax.experimental.pallas import tpu_sc as plsc

N = 10000
K = 15
T = 512
KP = 16
NPAD = 10240
NTILES = 32
SPT = NPAD // NTILES
CH = 4
NCH = SPT // CH
LANES = 16
NVC = T // LANES


def _sum_comps_body(comps_ref, out_ref):
    c = comps_ref[...]
    out_ref[...] = c[:, 0, :] + c[:, 1, :] + c[:, 2, :] + c[:, 3, :]


def _sum_components(comps):
    bn = 400
    grid = N // bn
    return pl.pallas_call(
        _sum_comps_body,
        grid=(grid,),
        in_specs=[pl.BlockSpec((bn, 4, T), lambda i: (i, 0, 0))],
        out_specs=pl.BlockSpec((bn, T), lambda i: (i, 0)),
        out_shape=jax.ShapeDtypeStruct((N, T), jnp.float32),
    )(comps)


def _dense_body(sp_ref, amp_ref, ph_ref, per_ref, tv_ref, out_ref):
    tv = tv_ref[...]
    inv_per = per_ref[...]
    amp = amp_ref[...]
    ph = ph_ref[...]
    sp = sp_ref[...]
    ang = (2.0 * np.pi) * tv / inv_per.reshape(8, 1)
    sinb = jnp.sin(ang)
    cosb = jnp.cos(ang)
    ones = jnp.ones_like(tv)
    basis = jnp.concatenate([ones, tv, sinb, cosb], axis=0)
    coef = jnp.concatenate(
        [sp[:, 0:1], sp[:, 1:2], amp * jnp.cos(ph), amp * jnp.sin(ph)], axis=1
    )
    out_ref[...] = jnp.dot(coef, basis, preferred_element_type=jnp.float32)


def _dense_signals(sp, amp_p, ph_p, per_p, tv):
    bn = 512
    grid = NPAD // bn
    return pl.pallas_call(
        _dense_body,
        grid=(grid,),
        in_specs=[
            pl.BlockSpec((bn, 8), lambda i: (i, 0)),
            pl.BlockSpec((bn, 8), lambda i: (i, 0)),
            pl.BlockSpec((bn, 8), lambda i: (i, 0)),
            pl.BlockSpec((1, 8), lambda i: (0, 0)),
            pl.BlockSpec((1, T), lambda i: (0, 0)),
        ],
        out_specs=pl.BlockSpec((bn, T), lambda i: (i, 0)),
        out_shape=jax.ShapeDtypeStruct((NPAD, T), jnp.float32),
    )(sp, amp_p, ph_p, per_p, tv)


def _sc_body(s_hbm, dense_hbm, idx_hbm, w_hbm, out_hbm,
             idx_v, w_v, rows_v, acc_v, gsem):
    nc = 2
    wid = lax.axis_index("s") * nc + lax.axis_index("c")
    base = wid * SPT
    pltpu.sync_copy(idx_hbm.at[pl.ds(wid * (SPT * KP), SPT * KP)], idx_v)
    pltpu.sync_copy(w_hbm.at[pl.ds(wid * (SPT * KP), SPT * KP)], w_v)

    gdn = lax.GatherDimensionNumbers(
        offset_dims=(), collapsed_slice_dims=(0,), start_index_map=(0,))

    def splat(vec, k):
        idxs = jnp.full((LANES,), k, jnp.int32)
        return lax.gather(vec, idxs[:, None], dimension_numbers=gdn,
                          slice_sizes=(1,),
                          mode=lax.GatherScatterMode.PROMISE_IN_BOUNDS)

    def chunk(c, carry):
        row0 = base + c * CH
        gcopy = pltpu.async_copy(
            s_hbm.at[idx_v.at[pl.ds(c * (CH * KP), CH * KP)]], rows_v, gsem)
        pltpu.sync_copy(dense_hbm.at[pl.ds(row0, CH)], acc_v)
        gcopy.wait()
        for s in range(CH):
            w_vec = w_v[pl.ds((c * CH + s) * KP, KP)]

            def kstep(k, kcarry):
                wk = splat(w_vec, k)
                for cc in range(NVC):
                    plsc.addupdate(
                        acc_v.at[s, pl.ds(cc * LANES, LANES)],
                        wk * rows_v[s * KP + k, pl.ds(cc * LANES, LANES)])
                return kcarry

            lax.fori_loop(0, KP, kstep, 0)
        pltpu.sync_copy(acc_v, out_hbm.at[pl.ds(row0, CH)])
        return carry

    lax.fori_loop(0, NCH, chunk, 0)


def _sc_gather(s_tab, dense, idx_flat, w_flat):
    mesh = plsc.VectorSubcoreMesh(core_axis_name="c", subcore_axis_name="s")
    return pl.kernel(
        _sc_body,
        mesh=mesh,
        out_type=jax.ShapeDtypeStruct((NPAD, T), jnp.float32),
        scratch_types=[
            pltpu.VMEM((SPT * KP,), jnp.int32),
            pltpu.VMEM((SPT * KP,), jnp.float32),
            pltpu.VMEM((CH * KP, T), jnp.float32),
            pltpu.VMEM((CH, T), jnp.float32),
            pltpu.SemaphoreType.DMA,
        ],
    )(s_tab, dense, idx_flat, w_flat)


@jax.jit
def kernel(time_vector, constant_offset, linear_trend, emd_seasonal_components,
           residual_amplitudes, residual_phases, residual_periods,
           emd_spatial_weights, local_spatial_weights,
           neighbor_indices, neighbor_weights, local_weights):
    f32 = jnp.float32
    sp = jnp.zeros((NPAD, 8), f32)
    sp = sp.at[:N, 0].set(constant_offset.astype(f32))
    sp = sp.at[:N, 1].set(linear_trend.astype(f32))
    amp_p = jnp.zeros((NPAD, 8), f32).at[:N, :5].set(residual_amplitudes.astype(f32))
    ph_p = jnp.zeros((NPAD, 8), f32).at[:N, :5].set(residual_phases.astype(f32))
    per_p = jnp.ones((1, 8), f32).at[0, :5].set(residual_periods.astype(f32))
    tv = time_vector.astype(f32).reshape(1, T)

    we = emd_spatial_weights.astype(f32)
    wl = local_spatial_weights.astype(f32)
    cw = wl[:, None] * local_weights.astype(f32) + we[:, None] * neighbor_weights.astype(f32)
    w16 = jnp.concatenate([cw, (1.0 - we - wl)[:, None]], axis=1)
    w_flat = jnp.zeros((NPAD, KP), f32).at[:N].set(w16).reshape(-1)

    idx16 = jnp.concatenate(
        [neighbor_indices.astype(jnp.int32),
         jnp.arange(N, dtype=jnp.int32)[:, None]], axis=1)
    idx_flat = jnp.zeros((NPAD, KP), jnp.int32).at[:N].set(idx16).reshape(-1)

    s_tab = _sum_components(emd_seasonal_components.astype(f32))
    dense = _dense_signals(sp, amp_p, ph_p, per_p, tv)
    out = _sc_gather(s_tab, dense, idx_flat, w_flat)
    return out[:N]


# speedup vs baseline: 2.5405x; 2.0720x over previous
import functools
import jax
import jax.numpy as jnp
import numpy as np
from jax import lax
from jax.experimental import pallas as pl
from jax.experimental.pallas import tpu as pltpu
from jax.experimental.pallas import tpu_sc as plsc

N = 10000
K = 15
T = 512
KP = 16
NPAD = 10240
NTILES = 32
SPT = NPAD // NTILES
CH = 4
NCH = SPT // CH
LANES = 16
NVC = T // LANES


def _sum_comps_body(comps_ref, out_ref):
    c = comps_ref[...]
    out_ref[...] = c[:, 0, :] + c[:, 1, :] + c[:, 2, :] + c[:, 3, :]


def _sum_components(comps):
    bn = 400
    grid = N // bn
    return pl.pallas_call(
        _sum_comps_body,
        grid=(grid,),
        in_specs=[pl.BlockSpec((bn, 4, T), lambda i: (i, 0, 0))],
        out_specs=pl.BlockSpec((bn, T), lambda i: (i, 0)),
        out_shape=jax.ShapeDtypeStruct((N, T), jnp.float32),
    )(comps)


def _dense_body(sp_ref, amp_ref, ph_ref, per_ref, tv_ref, out_ref):
    tv = tv_ref[...]
    inv_per = per_ref[...]
    amp = amp_ref[...]
    ph = ph_ref[...]
    sp = sp_ref[...]
    ang = (2.0 * np.pi) * tv / inv_per.reshape(8, 1)
    sinb = jnp.sin(ang)
    cosb = jnp.cos(ang)
    ones = jnp.ones_like(tv)
    basis = jnp.concatenate([ones, tv, sinb, cosb], axis=0)
    coef = jnp.concatenate(
        [sp[:, 0:1], sp[:, 1:2], amp * jnp.cos(ph), amp * jnp.sin(ph)], axis=1
    )
    out_ref[...] = jnp.dot(coef, basis, preferred_element_type=jnp.float32)


def _dense_signals(sp, amp_p, ph_p, per_p, tv):
    bn = 512
    grid = NPAD // bn
    return pl.pallas_call(
        _dense_body,
        grid=(grid,),
        in_specs=[
            pl.BlockSpec((bn, 8), lambda i: (i, 0)),
            pl.BlockSpec((bn, 8), lambda i: (i, 0)),
            pl.BlockSpec((bn, 8), lambda i: (i, 0)),
            pl.BlockSpec((1, 8), lambda i: (0, 0)),
            pl.BlockSpec((1, T), lambda i: (0, 0)),
        ],
        out_specs=pl.BlockSpec((bn, T), lambda i: (i, 0)),
        out_shape=jax.ShapeDtypeStruct((NPAD, T), jnp.float32),
    )(sp, amp_p, ph_p, per_p, tv)


def _sc_body(s_hbm, dense_hbm, idx_hbm, w_hbm, out_hbm,
             idx_v, w_v, rows_a, rows_b, acc_a, acc_b, sem_a, sem_b):
    nc = 2
    wid = lax.axis_index("s") * nc + lax.axis_index("c")
    base = wid * SPT
    pltpu.sync_copy(idx_hbm.at[pl.ds(wid * (SPT * KP), SPT * KP)], idx_v)
    pltpu.sync_copy(w_hbm.at[pl.ds(wid * (SPT * KP), SPT * KP)], w_v)

    gdn = lax.GatherDimensionNumbers(
        offset_dims=(), collapsed_slice_dims=(0,), start_index_map=(0,))

    def splat(vec, k):
        idxs = jnp.full((LANES,), k, jnp.int32)
        return lax.gather(vec, idxs[:, None], dimension_numbers=gdn,
                          slice_sizes=(1,),
                          mode=lax.GatherScatterMode.PROMISE_IN_BOUNDS)

    def gather_h(c, rows_ref, sem):
        return pltpu.make_async_copy(
            s_hbm.at[idx_v.at[pl.ds(c * (CH * KP), CH * KP)]], rows_ref, sem)

    def compute(c, rows_ref, acc_ref, sem):
        row0 = base + c * CH
        pltpu.sync_copy(dense_hbm.at[pl.ds(row0, CH)], acc_ref)
        gather_h(c, rows_ref, sem).wait()

        def station(s, carry):
            w_vec = w_v[pl.ds((c * CH + s) * KP, KP)]
            accs = [acc_ref[s, pl.ds(cc * LANES, LANES)] for cc in range(NVC)]
            for k in range(KP):
                wk = splat(w_vec, k)
                r = s * KP + k
                for cc in range(NVC):
                    accs[cc] = accs[cc] + wk * rows_ref[r, pl.ds(cc * LANES, LANES)]
            for cc in range(NVC):
                acc_ref[s, pl.ds(cc * LANES, LANES)] = accs[cc]
            return carry

        lax.fori_loop(0, CH, station, 0)
        pltpu.sync_copy(acc_ref, out_hbm.at[pl.ds(row0, CH)])

    gather_h(0, rows_a, sem_a).start()

    def pair(i, carry):
        c0 = i * 2
        gather_h(c0 + 1, rows_b, sem_b).start()
        compute(c0, rows_a, acc_a, sem_a)

        @pl.when(i < (NCH // 2) - 1)
        def _():
            gather_h(c0 + 2, rows_a, sem_a).start()

        compute(c0 + 1, rows_b, acc_b, sem_b)
        return carry

    lax.fori_loop(0, NCH // 2, pair, 0)


def _sc_gather(s_tab, dense, idx_flat, w_flat):
    mesh = plsc.VectorSubcoreMesh(core_axis_name="c", subcore_axis_name="s")
    return pl.kernel(
        _sc_body,
        mesh=mesh,
        out_type=jax.ShapeDtypeStruct((NPAD, T), jnp.float32),
        scratch_types=[
            pltpu.VMEM((SPT * KP,), jnp.int32),
            pltpu.VMEM((SPT * KP,), jnp.float32),
            pltpu.VMEM((CH * KP, T), jnp.float32),
            pltpu.VMEM((CH * KP, T), jnp.float32),
            pltpu.VMEM((CH, T), jnp.float32),
            pltpu.VMEM((CH, T), jnp.float32),
            pltpu.SemaphoreType.DMA,
            pltpu.SemaphoreType.DMA,
        ],
    )(s_tab, dense, idx_flat, w_flat)


@jax.jit
def kernel(time_vector, constant_offset, linear_trend, emd_seasonal_components,
           residual_amplitudes, residual_phases, residual_periods,
           emd_spatial_weights, local_spatial_weights,
           neighbor_indices, neighbor_weights, local_weights):
    f32 = jnp.float32
    sp = jnp.zeros((NPAD, 8), f32)
    sp = sp.at[:N, 0].set(constant_offset.astype(f32))
    sp = sp.at[:N, 1].set(linear_trend.astype(f32))
    amp_p = jnp.zeros((NPAD, 8), f32).at[:N, :5].set(residual_amplitudes.astype(f32))
    ph_p = jnp.zeros((NPAD, 8), f32).at[:N, :5].set(residual_phases.astype(f32))
    per_p = jnp.ones((1, 8), f32).at[0, :5].set(residual_periods.astype(f32))
    tv = time_vector.astype(f32).reshape(1, T)

    we = emd_spatial_weights.astype(f32)
    wl = local_spatial_weights.astype(f32)
    cw = wl[:, None] * local_weights.astype(f32) + we[:, None] * neighbor_weights.astype(f32)
    w16 = jnp.concatenate([cw, (1.0 - we - wl)[:, None]], axis=1)
    w_flat = jnp.zeros((NPAD, KP), f32).at[:N].set(w16).reshape(-1)

    idx16 = jnp.concatenate(
        [neighbor_indices.astype(jnp.int32),
         jnp.arange(N, dtype=jnp.int32)[:, None]], axis=1)
    idx_flat = jnp.zeros((NPAD, KP), jnp.int32).at[:N].set(idx16).reshape(-1)

    s_tab = _sum_components(emd_seasonal_components.astype(f32))
    dense = _dense_signals(sp, amp_p, ph_p, per_p, tv)
    out = _sc_gather(s_tab, dense, idx_flat, w_flat)
    return out[:N]


# speedup vs baseline: 2.5773x; 1.0145x over previous
import functools
import jax
import jax.numpy as jnp
import numpy as np
from jax import lax
from jax.experimental import pallas as pl
from jax.experimental.pallas import tpu as pltpu
from jax.experimental.pallas import tpu_sc as plsc

N = 10000
K = 15
T = 512
KP = 16
NPAD = 10240
NTILES = 32
SPT = NPAD // NTILES
CH = 4
NCH = SPT // CH
LANES = 16
NVC = T // LANES


def _sum_comps_body(comps_ref, out_ref):
    c = comps_ref[...]
    out_ref[...] = c[:, 0, :] + c[:, 1, :] + c[:, 2, :] + c[:, 3, :]


def _sum_components(comps):
    bn = 400
    grid = N // bn
    return pl.pallas_call(
        _sum_comps_body,
        grid=(grid,),
        in_specs=[pl.BlockSpec((bn, 4, T), lambda i: (i, 0, 0))],
        out_specs=pl.BlockSpec((bn, T), lambda i: (i, 0)),
        out_shape=jax.ShapeDtypeStruct((N, T), jnp.float32),
    )(comps)


def _dense_body(sp_ref, amp_ref, ph_ref, per_ref, tv_ref, out_ref):
    tv = tv_ref[...]
    inv_per = per_ref[...]
    amp = amp_ref[...]
    ph = ph_ref[...]
    sp = sp_ref[...]
    ang = (2.0 * np.pi) * tv / inv_per.reshape(8, 1)
    sinb = jnp.sin(ang)
    cosb = jnp.cos(ang)
    ones = jnp.ones_like(tv)
    basis = jnp.concatenate([ones, tv, sinb, cosb], axis=0)
    coef = jnp.concatenate(
        [sp[:, 0:1], sp[:, 1:2], amp * jnp.cos(ph), amp * jnp.sin(ph)], axis=1
    )
    out_ref[...] = jnp.dot(coef, basis, preferred_element_type=jnp.float32)


def _dense_signals(sp, amp_p, ph_p, per_p, tv):
    bn = 512
    grid = NPAD // bn
    return pl.pallas_call(
        _dense_body,
        grid=(grid,),
        in_specs=[
            pl.BlockSpec((bn, 8), lambda i: (i, 0)),
            pl.BlockSpec((bn, 8), lambda i: (i, 0)),
            pl.BlockSpec((bn, 8), lambda i: (i, 0)),
            pl.BlockSpec((1, 8), lambda i: (0, 0)),
            pl.BlockSpec((1, T), lambda i: (0, 0)),
        ],
        out_specs=pl.BlockSpec((bn, T), lambda i: (i, 0)),
        out_shape=jax.ShapeDtypeStruct((NPAD, T), jnp.float32),
    )(sp, amp_p, ph_p, per_p, tv)


def _sc_body(s_hbm, dense_hbm, idx_hbm, w_hbm, out_hbm,
             idx_v, w_v, rows_a, rows_b, acc_a, acc_b,
             sem_a, sem_b, dsem_a, dsem_b, osem_a, osem_b):
    nc = 2
    wid = lax.axis_index("s") * nc + lax.axis_index("c")
    base = wid * SPT
    pltpu.sync_copy(idx_hbm.at[pl.ds(wid * (SPT * KP), SPT * KP)], idx_v)
    pltpu.sync_copy(w_hbm.at[pl.ds(wid * (SPT * KP), SPT * KP)], w_v)

    gdn = lax.GatherDimensionNumbers(
        offset_dims=(), collapsed_slice_dims=(0,), start_index_map=(0,))

    def splat(vec, k):
        idxs = jnp.full((LANES,), k, jnp.int32)
        return lax.gather(vec, idxs[:, None], dimension_numbers=gdn,
                          slice_sizes=(1,),
                          mode=lax.GatherScatterMode.PROMISE_IN_BOUNDS)

    def gather_h(c, rows_ref, sem):
        return pltpu.make_async_copy(
            s_hbm.at[idx_v.at[pl.ds(c * (CH * KP), CH * KP)]], rows_ref, sem)

    def dense_h(c, acc_ref, sem):
        return pltpu.make_async_copy(
            dense_hbm.at[pl.ds(base + c * CH, CH)], acc_ref, sem)

    def out_h(c, acc_ref, sem):
        return pltpu.make_async_copy(
            acc_ref, out_hbm.at[pl.ds(base + c * CH, CH)], sem)

    def compute(c, rows_ref, acc_ref, sem, dsem, osem):
        dense_h(c, acc_ref, dsem).wait()
        gather_h(c, rows_ref, sem).wait()

        def station(s, carry):
            w_vec = w_v[pl.ds((c * CH + s) * KP, KP)]
            accs = [acc_ref[s, pl.ds(cc * LANES, LANES)] for cc in range(NVC)]
            for k in range(KP):
                wk = splat(w_vec, k)
                r = s * KP + k
                for cc in range(NVC):
                    accs[cc] = accs[cc] + wk * rows_ref[r, pl.ds(cc * LANES, LANES)]
            for cc in range(NVC):
                acc_ref[s, pl.ds(cc * LANES, LANES)] = accs[cc]
            return carry

        lax.fori_loop(0, CH, station, 0)
        out_h(c, acc_ref, osem).start()

    gather_h(0, rows_a, sem_a).start()
    dense_h(0, acc_a, dsem_a).start()
    npair = NCH // 2

    def pair(i, carry):
        c0 = i * 2
        gather_h(c0 + 1, rows_b, sem_b).start()
        compute(c0, rows_a, acc_a, sem_a, dsem_a, osem_a)

        @pl.when(i < npair - 1)
        def _():
            gather_h(c0 + 2, rows_a, sem_a).start()

        @pl.when(i > 0)
        def _():
            out_h(0, acc_b, osem_b).wait()
        dense_h(c0 + 1, acc_b, dsem_b).start()
        compute(c0 + 1, rows_b, acc_b, sem_b, dsem_b, osem_b)

        @pl.when(i < npair - 1)
        def _():
            out_h(0, acc_a, osem_a).wait()
            dense_h(c0 + 2, acc_a, dsem_a).start()

        return carry

    lax.fori_loop(0, npair, pair, 0)
    out_h(0, acc_a, osem_a).wait()
    out_h(0, acc_b, osem_b).wait()


def _sc_gather(s_tab, dense, idx_flat, w_flat):
    mesh = plsc.VectorSubcoreMesh(core_axis_name="c", subcore_axis_name="s")
    return pl.kernel(
        _sc_body,
        mesh=mesh,
        out_type=jax.ShapeDtypeStruct((NPAD, T), jnp.float32),
        scratch_types=[
            pltpu.VMEM((SPT * KP,), jnp.int32),
            pltpu.VMEM((SPT * KP,), jnp.float32),
            pltpu.VMEM((CH * KP, T), jnp.float32),
            pltpu.VMEM((CH * KP, T), jnp.float32),
            pltpu.VMEM((CH, T), jnp.float32),
            pltpu.VMEM((CH, T), jnp.float32),
            pltpu.SemaphoreType.DMA,
            pltpu.SemaphoreType.DMA,
            pltpu.SemaphoreType.DMA,
            pltpu.SemaphoreType.DMA,
            pltpu.SemaphoreType.DMA,
            pltpu.SemaphoreType.DMA,
        ],
    )(s_tab, dense, idx_flat, w_flat)


@jax.jit
def kernel(time_vector, constant_offset, linear_trend, emd_seasonal_components,
           residual_amplitudes, residual_phases, residual_periods,
           emd_spatial_weights, local_spatial_weights,
           neighbor_indices, neighbor_weights, local_weights):
    f32 = jnp.float32
    sp = jnp.zeros((NPAD, 8), f32)
    sp = sp.at[:N, 0].set(constant_offset.astype(f32))
    sp = sp.at[:N, 1].set(linear_trend.astype(f32))
    amp_p = jnp.zeros((NPAD, 8), f32).at[:N, :5].set(residual_amplitudes.astype(f32))
    ph_p = jnp.zeros((NPAD, 8), f32).at[:N, :5].set(residual_phases.astype(f32))
    per_p = jnp.ones((1, 8), f32).at[0, :5].set(residual_periods.astype(f32))
    tv = time_vector.astype(f32).reshape(1, T)

    we = emd_spatial_weights.astype(f32)
    wl = local_spatial_weights.astype(f32)
    cw = wl[:, None] * local_weights.astype(f32) + we[:, None] * neighbor_weights.astype(f32)
    w16 = jnp.concatenate([cw, (1.0 - we - wl)[:, None]], axis=1)
    w_flat = jnp.zeros((NPAD, KP), f32).at[:N].set(w16).reshape(-1)

    idx16 = jnp.concatenate(
        [neighbor_indices.astype(jnp.int32),
         jnp.arange(N, dtype=jnp.int32)[:, None]], axis=1)
    idx_flat = jnp.zeros((NPAD, KP), jnp.int32).at[:N].set(idx16).reshape(-1)

    s_tab = _sum_components(emd_seasonal_components.astype(f32))
    dense = _dense_signals(sp, amp_p, ph_p, per_p, tv)
    out = _sc_gather(s_tab, dense, idx_flat, w_flat)
    return out[:N]


# speedup vs baseline: 3.0191x; 1.1714x over previous
import functools
import jax
import jax.numpy as jnp
import numpy as np
from jax import lax
from jax.experimental import pallas as pl
from jax.experimental.pallas import tpu as pltpu
from jax.experimental.pallas import tpu_sc as plsc

N = 10000
K = 15
T = 512
KP = 16
NPAD = 10240
NTILES = 32
SPT = NPAD // NTILES
CH = 8
NCH = SPT // CH
LANES = 16
NVC = T // LANES
TW = T // 2


def _sum_comps_body(comps_ref, out_ref):
    c = comps_ref[...]
    out_ref[...] = c[:, 0, :] + c[:, 1, :] + c[:, 2, :] + c[:, 3, :]


def _sum_components(comps):
    bn = 400
    grid = N // bn
    return pl.pallas_call(
        _sum_comps_body,
        grid=(grid,),
        in_specs=[pl.BlockSpec((bn, 4, T), lambda i: (i, 0, 0))],
        out_specs=pl.BlockSpec((bn, T), lambda i: (i, 0)),
        out_shape=jax.ShapeDtypeStruct((N, T), jnp.float32),
    )(comps)


def _dense_body(sp_ref, amp_ref, ph_ref, per_ref, tv_ref, out_ref):
    tv = tv_ref[...]
    inv_per = per_ref[...]
    amp = amp_ref[...]
    ph = ph_ref[...]
    sp = sp_ref[...]
    ang = (2.0 * np.pi) * tv / inv_per.reshape(8, 1)
    sinb = jnp.sin(ang)
    cosb = jnp.cos(ang)
    ones = jnp.ones_like(tv)
    basis = jnp.concatenate([ones, tv, sinb, cosb], axis=0)
    coef = jnp.concatenate(
        [sp[:, 0:1], sp[:, 1:2], amp * jnp.cos(ph), amp * jnp.sin(ph)], axis=1
    )
    out_ref[...] = jnp.dot(coef, basis, preferred_element_type=jnp.float32)


def _dense_signals(sp, amp_p, ph_p, per_p, tv):
    bn = 512
    grid = NPAD // bn
    return pl.pallas_call(
        _dense_body,
        grid=(grid,),
        in_specs=[
            pl.BlockSpec((bn, 8), lambda i: (i, 0)),
            pl.BlockSpec((bn, 8), lambda i: (i, 0)),
            pl.BlockSpec((bn, 8), lambda i: (i, 0)),
            pl.BlockSpec((1, 8), lambda i: (0, 0)),
            pl.BlockSpec((1, T), lambda i: (0, 0)),
        ],
        out_specs=pl.BlockSpec((bn, T), lambda i: (i, 0)),
        out_shape=jax.ShapeDtypeStruct((NPAD, T), jnp.float32),
    )(sp, amp_p, ph_p, per_p, tv)


def _sc_body(s_hbm, dense_hbm, idx_hbm, w_hbm, out_hbm,
             idx_v, w_v, rows_a, rows_b, acc_a, acc_b,
             sem_a, sem_b, dsem_a, dsem_b, osem_a, osem_b):
    nc = 2
    wid = lax.axis_index("s") * nc + lax.axis_index("c")
    base = wid * SPT
    pltpu.sync_copy(idx_hbm.at[pl.ds(wid * (SPT * KP), SPT * KP)], idx_v)
    pltpu.sync_copy(w_hbm.at[pl.ds(wid * (SPT * KP), SPT * KP)], w_v)

    gdn = lax.GatherDimensionNumbers(
        offset_dims=(), collapsed_slice_dims=(0,), start_index_map=(0,))

    def splat(vec, k):
        idxs = jnp.full((LANES,), k, jnp.int32)
        return lax.gather(vec, idxs[:, None], dimension_numbers=gdn,
                          slice_sizes=(1,),
                          mode=lax.GatherScatterMode.PROMISE_IN_BOUNDS)

    def gather_h(c, rows_ref, sem):
        return pltpu.make_async_copy(
            s_hbm.at[idx_v.at[pl.ds(c * (CH * KP), CH * KP)]], rows_ref, sem)

    def dense_h(c, acc_ref, sem):
        return pltpu.make_async_copy(
            dense_hbm.at[pl.ds(base + c * CH, CH)], acc_ref, sem)

    def out_h(c, acc_ref, sem):
        return pltpu.make_async_copy(
            acc_ref, out_hbm.at[pl.ds(base + c * CH, CH)], sem)

    himask = jnp.int32(-65536)

    def compute(c, rows_ref, acc_ref, sem, dsem, osem):
        dense_h(c, acc_ref, dsem).wait()
        gather_h(c, rows_ref, sem).wait()

        def station(s, carry):
            w_vec = w_v[pl.ds((c * CH + s) * KP, KP)]
            accs = [acc_ref[s, pl.ds(cc * LANES, LANES)] for cc in range(NVC)]
            for k in range(KP):
                wk = splat(w_vec, k)
                r = s * KP + k
                for cw in range(TW // LANES):
                    w32 = rows_ref[r, pl.ds(cw * LANES, LANES)]
                    lo = plsc.bitcast(w32 << 16, jnp.float32)
                    hi = plsc.bitcast(w32 & himask, jnp.float32)
                    accs[2 * cw] = accs[2 * cw] + wk * lo
                    accs[2 * cw + 1] = accs[2 * cw + 1] + wk * hi
            for cc in range(NVC):
                acc_ref[s, pl.ds(cc * LANES, LANES)] = accs[cc]
            return carry

        lax.fori_loop(0, CH, station, 0)
        out_h(c, acc_ref, osem).start()

    gather_h(0, rows_a, sem_a).start()
    dense_h(0, acc_a, dsem_a).start()
    npair = NCH // 2

    def pair(i, carry):
        c0 = i * 2
        gather_h(c0 + 1, rows_b, sem_b).start()
        compute(c0, rows_a, acc_a, sem_a, dsem_a, osem_a)

        @pl.when(i < npair - 1)
        def _():
            gather_h(c0 + 2, rows_a, sem_a).start()

        @pl.when(i > 0)
        def _():
            out_h(0, acc_b, osem_b).wait()
        dense_h(c0 + 1, acc_b, dsem_b).start()
        compute(c0 + 1, rows_b, acc_b, sem_b, dsem_b, osem_b)

        @pl.when(i < npair - 1)
        def _():
            out_h(0, acc_a, osem_a).wait()
            dense_h(c0 + 2, acc_a, dsem_a).start()

        return carry

    lax.fori_loop(0, npair, pair, 0)
    out_h(0, acc_a, osem_a).wait()
    out_h(0, acc_b, osem_b).wait()


def _sc_gather(s_tab, dense, idx_flat, w_flat):
    mesh = plsc.VectorSubcoreMesh(core_axis_name="c", subcore_axis_name="s")
    return pl.kernel(
        _sc_body,
        mesh=mesh,
        compiler_params=pltpu.CompilerParams(needs_layout_passes=False),
        out_type=jax.ShapeDtypeStruct((NPAD, T), jnp.float32),
        scratch_types=[
            pltpu.VMEM((SPT * KP,), jnp.int32),
            pltpu.VMEM((SPT * KP,), jnp.float32),
            pltpu.VMEM((CH * KP, TW), jnp.int32),
            pltpu.VMEM((CH * KP, TW), jnp.int32),
            pltpu.VMEM((CH, T), jnp.float32),
            pltpu.VMEM((CH, T), jnp.float32),
            pltpu.SemaphoreType.DMA,
            pltpu.SemaphoreType.DMA,
            pltpu.SemaphoreType.DMA,
            pltpu.SemaphoreType.DMA,
            pltpu.SemaphoreType.DMA,
            pltpu.SemaphoreType.DMA,
        ],
    )(s_tab, dense, idx_flat, w_flat)


@jax.jit
def kernel(time_vector, constant_offset, linear_trend, emd_seasonal_components,
           residual_amplitudes, residual_phases, residual_periods,
           emd_spatial_weights, local_spatial_weights,
           neighbor_indices, neighbor_weights, local_weights):
    f32 = jnp.float32
    sp = jnp.zeros((NPAD, 8), f32)
    sp = sp.at[:N, 0].set(constant_offset.astype(f32))
    sp = sp.at[:N, 1].set(linear_trend.astype(f32))
    amp_p = jnp.zeros((NPAD, 8), f32).at[:N, :5].set(residual_amplitudes.astype(f32))
    ph_p = jnp.zeros((NPAD, 8), f32).at[:N, :5].set(residual_phases.astype(f32))
    per_p = jnp.ones((1, 8), f32).at[0, :5].set(residual_periods.astype(f32))
    tv = time_vector.astype(f32).reshape(1, T)

    we = emd_spatial_weights.astype(f32)
    wl = local_spatial_weights.astype(f32)
    cw = wl[:, None] * local_weights.astype(f32) + we[:, None] * neighbor_weights.astype(f32)
    w16 = jnp.concatenate([cw, (1.0 - we - wl)[:, None]], axis=1)
    w_flat = jnp.zeros((NPAD, KP), f32).at[:N].set(w16).reshape(-1)

    idx16 = jnp.concatenate(
        [neighbor_indices.astype(jnp.int32),
         jnp.arange(N, dtype=jnp.int32)[:, None]], axis=1)
    idx_flat = jnp.zeros((NPAD, KP), jnp.int32).at[:N].set(idx16).reshape(-1)

    s_tab = _sum_components(emd_seasonal_components.astype(f32))
    s_packed = jax.lax.bitcast_convert_type(
        s_tab.astype(jnp.bfloat16).reshape(N, 16, 2, 16)
        .transpose(0, 1, 3, 2).reshape(N, TW, 2), jnp.int32)
    dense = _dense_signals(sp, amp_p, ph_p, per_p, tv)
    out = _sc_gather(s_packed, dense, idx_flat, w_flat)
    return out[:N]


# speedup vs baseline: 3.0394x; 1.0067x over previous
import functools
import jax
import jax.numpy as jnp
import numpy as np
from jax import lax
from jax.experimental import pallas as pl
from jax.experimental.pallas import tpu as pltpu
from jax.experimental.pallas import tpu_sc as plsc

N = 10000
K = 15
T = 512
KP = 16
NPAD = 10240
NTILES = 32
SPT = NPAD // NTILES
CH = 8
NCH = SPT // CH
LANES = 16
NVC = T // LANES
TW = T // 2


def _sum_comps_body(comps_ref, out_ref):
    c = comps_ref[...]
    out_ref[...] = c[:, 0, :] + c[:, 1, :] + c[:, 2, :] + c[:, 3, :]


def _sum_components(comps):
    bn = 400
    grid = N // bn
    return pl.pallas_call(
        _sum_comps_body,
        grid=(grid,),
        in_specs=[pl.BlockSpec((bn, 4, T), lambda i: (i, 0, 0))],
        out_specs=pl.BlockSpec((bn, T), lambda i: (i, 0)),
        out_shape=jax.ShapeDtypeStruct((N, T), jnp.float32),
    )(comps)


def _dense_body(sp_ref, amp_ref, ph_ref, per_ref, tv_ref, out_ref):
    tv = tv_ref[...]
    inv_per = per_ref[...]
    amp = amp_ref[...]
    ph = ph_ref[...]
    sp = sp_ref[...]
    ang = (2.0 * np.pi) * tv / inv_per.reshape(8, 1)
    sinb = jnp.sin(ang)
    cosb = jnp.cos(ang)
    ones = jnp.ones_like(tv)
    basis = jnp.concatenate([ones, tv, sinb, cosb], axis=0)
    coef = jnp.concatenate(
        [sp[:, 0:1], sp[:, 1:2], amp * jnp.cos(ph), amp * jnp.sin(ph)], axis=1
    )
    out_ref[...] = jnp.dot(coef, basis, preferred_element_type=jnp.float32)


def _dense_signals(sp, amp_p, ph_p, per_p, tv):
    bn = 512
    grid = NPAD // bn
    return pl.pallas_call(
        _dense_body,
        grid=(grid,),
        in_specs=[
            pl.BlockSpec((bn, 8), lambda i: (i, 0)),
            pl.BlockSpec((bn, 8), lambda i: (i, 0)),
            pl.BlockSpec((bn, 8), lambda i: (i, 0)),
            pl.BlockSpec((1, 8), lambda i: (0, 0)),
            pl.BlockSpec((1, T), lambda i: (0, 0)),
        ],
        out_specs=pl.BlockSpec((bn, T), lambda i: (i, 0)),
        out_shape=jax.ShapeDtypeStruct((NPAD, T), jnp.float32),
    )(sp, amp_p, ph_p, per_p, tv)


def _sc_body(s_hbm, dense_hbm, idx_hbm, w_hbm, out_hbm,
             idx_v, w_v, rows_a, rows_b, acc_a, acc_b,
             sem_a, sem_b, dsem_a, dsem_b, osem_a, osem_b):
    nc = 2
    wid = lax.axis_index("s") * nc + lax.axis_index("c")
    base = wid * SPT
    pltpu.sync_copy(idx_hbm.at[pl.ds(wid * (SPT * KP), SPT * KP)], idx_v)
    pltpu.sync_copy(w_hbm.at[pl.ds(wid * (SPT * KP), SPT * KP)], w_v)

    gdn = lax.GatherDimensionNumbers(
        offset_dims=(), collapsed_slice_dims=(0,), start_index_map=(0,))

    def splat(vec, k):
        idxs = jnp.full((LANES,), k, jnp.int32)
        return lax.gather(vec, idxs[:, None], dimension_numbers=gdn,
                          slice_sizes=(1,),
                          mode=lax.GatherScatterMode.PROMISE_IN_BOUNDS)

    def gather_h(c, rows_ref, sem):
        return pltpu.make_async_copy(
            s_hbm.at[idx_v.at[pl.ds(c * (CH * KP), CH * KP)]], rows_ref, sem)

    def dense_h(c, acc_ref, sem):
        return pltpu.make_async_copy(
            dense_hbm.at[pl.ds(base + c * CH, CH)], acc_ref, sem)

    def out_h(c, acc_ref, sem):
        return pltpu.make_async_copy(
            acc_ref, out_hbm.at[pl.ds(base + c * CH, CH)], sem)

    def compute(c, rows_ref, acc_ref, sem, dsem, osem):
        dense_h(c, acc_ref, dsem).wait()
        gather_h(c, rows_ref, sem).wait()

        def station(s, carry):
            w_vec = w_v[pl.ds((c * CH + s) * KP, KP)]
            accs = [None] * (TW // LANES)
            for k in range(KP):
                wk = plsc.bitcast(splat(w_vec, k), jnp.bfloat16)
                r = s * KP + k
                for cw in range(TW // LANES):
                    rb = plsc.bitcast(rows_ref[r, pl.ds(cw * LANES, LANES)],
                                      jnp.bfloat16)
                    p = wk * rb
                    accs[cw] = p if k == 0 else accs[cw] + p
            for cw in range(TW // LANES):
                lo, hi = plsc.unpack(accs[cw], format=plsc.PackFormat.INTERLEAVED)
                plsc.addupdate(acc_ref.at[s, pl.ds(2 * cw * LANES, LANES)], lo)
                plsc.addupdate(acc_ref.at[s, pl.ds((2 * cw + 1) * LANES, LANES)], hi)
            return carry

        lax.fori_loop(0, CH, station, 0)
        out_h(c, acc_ref, osem).start()

    gather_h(0, rows_a, sem_a).start()
    dense_h(0, acc_a, dsem_a).start()
    npair = NCH // 2

    def pair(i, carry):
        c0 = i * 2
        gather_h(c0 + 1, rows_b, sem_b).start()
        compute(c0, rows_a, acc_a, sem_a, dsem_a, osem_a)

        @pl.when(i < npair - 1)
        def _():
            gather_h(c0 + 2, rows_a, sem_a).start()

        @pl.when(i > 0)
        def _():
            out_h(0, acc_b, osem_b).wait()
        dense_h(c0 + 1, acc_b, dsem_b).start()
        compute(c0 + 1, rows_b, acc_b, sem_b, dsem_b, osem_b)

        @pl.when(i < npair - 1)
        def _():
            out_h(0, acc_a, osem_a).wait()
            dense_h(c0 + 2, acc_a, dsem_a).start()

        return carry

    lax.fori_loop(0, npair, pair, 0)
    out_h(0, acc_a, osem_a).wait()
    out_h(0, acc_b, osem_b).wait()


def _sc_gather(s_tab, dense, idx_flat, w_flat):
    mesh = plsc.VectorSubcoreMesh(core_axis_name="c", subcore_axis_name="s")
    return pl.kernel(
        _sc_body,
        mesh=mesh,
        compiler_params=pltpu.CompilerParams(needs_layout_passes=False),
        out_type=jax.ShapeDtypeStruct((NPAD, T), jnp.float32),
        scratch_types=[
            pltpu.VMEM((SPT * KP,), jnp.int32),
            pltpu.VMEM((SPT * KP,), jnp.int32),
            pltpu.VMEM((CH * KP, TW), jnp.int32),
            pltpu.VMEM((CH * KP, TW), jnp.int32),
            pltpu.VMEM((CH, T), jnp.float32),
            pltpu.VMEM((CH, T), jnp.float32),
            pltpu.SemaphoreType.DMA,
            pltpu.SemaphoreType.DMA,
            pltpu.SemaphoreType.DMA,
            pltpu.SemaphoreType.DMA,
            pltpu.SemaphoreType.DMA,
            pltpu.SemaphoreType.DMA,
        ],
    )(s_tab, dense, idx_flat, w_flat)


@jax.jit
def kernel(time_vector, constant_offset, linear_trend, emd_seasonal_components,
           residual_amplitudes, residual_phases, residual_periods,
           emd_spatial_weights, local_spatial_weights,
           neighbor_indices, neighbor_weights, local_weights):
    f32 = jnp.float32
    sp = jnp.zeros((NPAD, 8), f32)
    sp = sp.at[:N, 0].set(constant_offset.astype(f32))
    sp = sp.at[:N, 1].set(linear_trend.astype(f32))
    amp_p = jnp.zeros((NPAD, 8), f32).at[:N, :5].set(residual_amplitudes.astype(f32))
    ph_p = jnp.zeros((NPAD, 8), f32).at[:N, :5].set(residual_phases.astype(f32))
    per_p = jnp.ones((1, 8), f32).at[0, :5].set(residual_periods.astype(f32))
    tv = time_vector.astype(f32).reshape(1, T)

    we = emd_spatial_weights.astype(f32)
    wl = local_spatial_weights.astype(f32)
    cw = wl[:, None] * local_weights.astype(f32) + we[:, None] * neighbor_weights.astype(f32)
    w16 = jnp.concatenate([cw, (1.0 - we - wl)[:, None]], axis=1)
    wbits = jax.lax.bitcast_convert_type(
        w16.astype(jnp.bfloat16), jnp.uint16).astype(jnp.int32)
    w_pair = wbits | (wbits << 16)
    w_flat = jnp.zeros((NPAD, KP), jnp.int32).at[:N].set(w_pair).reshape(-1)

    idx16 = jnp.concatenate(
        [neighbor_indices.astype(jnp.int32),
         jnp.arange(N, dtype=jnp.int32)[:, None]], axis=1)
    idx_flat = jnp.zeros((NPAD, KP), jnp.int32).at[:N].set(idx16).reshape(-1)

    s_tab = _sum_components(emd_seasonal_components.astype(f32))
    s_packed = jax.lax.bitcast_convert_type(
        s_tab.astype(jnp.bfloat16).reshape(N, 16, 2, 16)
        .transpose(0, 1, 3, 2).reshape(N, TW, 2), jnp.int32)
    dense = _dense_signals(sp, amp_p, ph_p, per_p, tv)
    out = _sc_gather(s_packed, dense, idx_flat, w_flat)
    return out[:N]


# speedup vs baseline: 5.2193x; 1.7172x over previous
import functools
import jax
import jax.numpy as jnp
import numpy as np
from jax import lax
from jax.experimental import pallas as pl
from jax.experimental.pallas import tpu as pltpu
from jax.experimental.pallas import tpu_sc as plsc

N = 10000
K = 15
T = 512
KP = 16
NTILES = 32
SPT = 320
NLAST = N - (NTILES - 1) * SPT
CH = 8
NCH = SPT // CH
NCH_LAST = NLAST // CH
LANES = 16
TW = T // 2
HALF = T // 2


def _sum_pack_body(comps_ref, out_ref):
    c = comps_ref[...]
    s = c[:, 0, :] + c[:, 1, :] + c[:, 2, :] + c[:, 3, :]
    u = lax.bitcast_convert_type(s, jnp.uint32)
    r = u + jnp.uint32(0x7FFF) + ((u >> jnp.uint32(16)) & jnp.uint32(1))
    ra = r[:, :HALF]
    rb = r[:, HALF:]
    word = (ra >> jnp.uint32(16)) | (rb & jnp.uint32(0xFFFF0000))
    out_ref[...] = lax.bitcast_convert_type(word, jnp.int32)


def _sum_pack(comps):
    bn = 400
    grid = N // bn
    return pl.pallas_call(
        _sum_pack_body,
        grid=(grid,),
        in_specs=[pl.BlockSpec((bn, 4, T), lambda i: (i, 0, 0))],
        out_specs=pl.BlockSpec((bn, TW), lambda i: (i, 0)),
        out_shape=jax.ShapeDtypeStruct((N, TW), jnp.int32),
    )(comps)


def _dense_body(off_ref, tr_ref, amp_ref, ph_ref, per_ref, tv_ref, out_ref):
    tv = tv_ref[...]
    per = per_ref[...]
    amp = amp_ref[...]
    ph = ph_ref[...]
    off = off_ref[...]
    tr = tr_ref[...]
    ang = (2.0 * np.pi) * tv / per.reshape(8, 1)
    sinb = jnp.sin(ang)
    cosb = jnp.cos(ang)
    base = off + tr * tv
    out_ref[...] = (
        base
        + jnp.dot(amp * jnp.cos(ph), sinb, preferred_element_type=jnp.float32)
        + jnp.dot(amp * jnp.sin(ph), cosb, preferred_element_type=jnp.float32)
    )


def _dense_signals(off, tr, amp_p, ph_p, per_p, tv):
    bn = 400
    grid = N // bn
    return pl.pallas_call(
        _dense_body,
        grid=(grid,),
        in_specs=[
            pl.BlockSpec((bn, 1), lambda i: (i, 0)),
            pl.BlockSpec((bn, 1), lambda i: (i, 0)),
            pl.BlockSpec((bn, 8), lambda i: (i, 0)),
            pl.BlockSpec((bn, 8), lambda i: (i, 0)),
            pl.BlockSpec((1, 8), lambda i: (0, 0)),
            pl.BlockSpec((1, T), lambda i: (0, 0)),
        ],
        out_specs=pl.BlockSpec((bn, T), lambda i: (i, 0)),
        out_shape=jax.ShapeDtypeStruct((N, T), jnp.float32),
    )(off, tr, amp_p, ph_p, per_p, tv)


def _sc_body(s_hbm, dense_hbm, idx_hbm, w_hbm, out_hbm,
             idx_v, w_v, rows_a, rows_b, acc_a, acc_b,
             sem_a, sem_b, dsem_a, dsem_b, osem_a, osem_b):
    nc = 2
    wid = lax.axis_index("s") * nc + lax.axis_index("c")
    base = wid * SPT
    last = wid == NTILES - 1

    @pl.when(last)
    def _():
        pltpu.sync_copy(idx_hbm.at[pl.ds(base * KP, NLAST * KP)],
                        idx_v.at[pl.ds(0, NLAST * KP)])
        pltpu.sync_copy(w_hbm.at[pl.ds(base * KP, NLAST * KP)],
                        w_v.at[pl.ds(0, NLAST * KP)])

    @pl.when(jnp.logical_not(last))
    def _():
        pltpu.sync_copy(idx_hbm.at[pl.ds(base * KP, SPT * KP)], idx_v)
        pltpu.sync_copy(w_hbm.at[pl.ds(base * KP, SPT * KP)], w_v)

    nch = jnp.where(last, NCH_LAST, NCH)

    gdn = lax.GatherDimensionNumbers(
        offset_dims=(), collapsed_slice_dims=(0,), start_index_map=(0,))

    def splat(vec, k):
        idxs = jnp.full((LANES,), k, jnp.int32)
        return lax.gather(vec, idxs[:, None], dimension_numbers=gdn,
                          slice_sizes=(1,),
                          mode=lax.GatherScatterMode.PROMISE_IN_BOUNDS)

    def gather_h(c, rows_ref, sem):
        return pltpu.make_async_copy(
            s_hbm.at[idx_v.at[pl.ds(c * (CH * KP), CH * KP)]], rows_ref, sem)

    def dense_h(c, acc_ref, sem):
        return pltpu.make_async_copy(
            dense_hbm.at[pl.ds(base + c * CH, CH)], acc_ref, sem)

    def out_h(c, acc_ref, sem):
        return pltpu.make_async_copy(
            acc_ref, out_hbm.at[pl.ds(base + c * CH, CH)], sem)

    def compute(c, rows_ref, acc_ref, sem, dsem, osem):
        dense_h(c, acc_ref, dsem).wait()
        gather_h(c, rows_ref, sem).wait()

        def station(s, carry):
            w_vec = w_v[pl.ds((c * CH + s) * KP, KP)]
            accs = [None] * (TW // LANES)
            for k in range(KP):
                wk = plsc.bitcast(splat(w_vec, k), jnp.bfloat16)
                r = s * KP + k
                for cw in range(TW // LANES):
                    rb = plsc.bitcast(rows_ref[r, pl.ds(cw * LANES, LANES)],
                                      jnp.bfloat16)
                    p = wk * rb
                    accs[cw] = p if k == 0 else accs[cw] + p
            for cw in range(TW // LANES):
                lo, hi = plsc.unpack(accs[cw], format=plsc.PackFormat.INTERLEAVED)
                plsc.addupdate(acc_ref.at[s, pl.ds(cw * LANES, LANES)], lo)
                plsc.addupdate(acc_ref.at[s, pl.ds(HALF + cw * LANES, LANES)], hi)
            return carry

        lax.fori_loop(0, CH, station, 0)
        out_h(c, acc_ref, osem).start()

    gather_h(0, rows_a, sem_a).start()
    dense_h(0, acc_a, dsem_a).start()
    npair = nch // 2

    def pair(i, carry):
        c0 = i * 2
        gather_h(c0 + 1, rows_b, sem_b).start()
        compute(c0, rows_a, acc_a, sem_a, dsem_a, osem_a)

        @pl.when(i < npair - 1)
        def _():
            gather_h(c0 + 2, rows_a, sem_a).start()

        @pl.when(i > 0)
        def _():
            out_h(0, acc_b, osem_b).wait()
        dense_h(c0 + 1, acc_b, dsem_b).start()
        compute(c0 + 1, rows_b, acc_b, sem_b, dsem_b, osem_b)

        @pl.when(i < npair - 1)
        def _():
            out_h(0, acc_a, osem_a).wait()
            dense_h(c0 + 2, acc_a, dsem_a).start()

        return carry

    lax.fori_loop(0, npair, pair, 0)
    out_h(0, acc_a, osem_a).wait()
    out_h(0, acc_b, osem_b).wait()


def _sc_gather(s_tab, dense, idx_flat, w_flat):
    mesh = plsc.VectorSubcoreMesh(core_axis_name="c", subcore_axis_name="s")
    return pl.kernel(
        _sc_body,
        mesh=mesh,
        compiler_params=pltpu.CompilerParams(needs_layout_passes=False),
        out_type=jax.ShapeDtypeStruct((N, T), jnp.float32),
        scratch_types=[
            pltpu.VMEM((SPT * KP,), jnp.int32),
            pltpu.VMEM((SPT * KP,), jnp.int32),
            pltpu.VMEM((CH * KP, TW), jnp.int32),
            pltpu.VMEM((CH * KP, TW), jnp.int32),
            pltpu.VMEM((CH, T), jnp.float32),
            pltpu.VMEM((CH, T), jnp.float32),
            pltpu.SemaphoreType.DMA,
            pltpu.SemaphoreType.DMA,
            pltpu.SemaphoreType.DMA,
            pltpu.SemaphoreType.DMA,
            pltpu.SemaphoreType.DMA,
            pltpu.SemaphoreType.DMA,
        ],
    )(s_tab, dense, idx_flat, w_flat)


@jax.jit
def kernel(time_vector, constant_offset, linear_trend, emd_seasonal_components,
           residual_amplitudes, residual_phases, residual_periods,
           emd_spatial_weights, local_spatial_weights,
           neighbor_indices, neighbor_weights, local_weights):
    f32 = jnp.float32
    amp_p = jnp.zeros((N, 8), f32).at[:, :5].set(residual_amplitudes.astype(f32))
    ph_p = jnp.zeros((N, 8), f32).at[:, :5].set(residual_phases.astype(f32))
    per_p = jnp.ones((1, 8), f32).at[0, :5].set(residual_periods.astype(f32))
    tv = time_vector.astype(f32).reshape(1, T)
    off = constant_offset.astype(f32).reshape(N, 1)
    tr = linear_trend.astype(f32).reshape(N, 1)

    we = emd_spatial_weights.astype(f32)
    wl = local_spatial_weights.astype(f32)
    cw = wl[:, None] * local_weights.astype(f32) + we[:, None] * neighbor_weights.astype(f32)
    w16 = jnp.concatenate([cw, (1.0 - we - wl)[:, None]], axis=1)
    wbits = lax.bitcast_convert_type(
        w16.astype(jnp.bfloat16), jnp.uint16).astype(jnp.int32)
    w_flat = (wbits | (wbits << 16)).reshape(-1)

    idx_flat = jnp.concatenate(
        [neighbor_indices.astype(jnp.int32),
         jnp.arange(N, dtype=jnp.int32)[:, None]], axis=1).reshape(-1)

    s_packed = _sum_pack(emd_seasonal_components.astype(f32))
    dense = _dense_signals(off, tr, amp_p, ph_p, per_p, tv)
    return _sc_gather(s_packed, dense, idx_flat, w_flat)


# speedup vs baseline: 5.8781x; 1.1262x over previous
import functools
import jax
import jax.numpy as jnp
import numpy as np
from jax import lax
from jax.experimental import pallas as pl
from jax.experimental.pallas import tpu as pltpu
from jax.experimental.pallas import tpu_sc as plsc

N = 10000
K = 15
T = 512
KP = 16
NTILES = 32
SPT = 320
NLAST = N - (NTILES - 1) * SPT
CH = 8
NCH = SPT // CH
NCH_LAST = NLAST // CH
LANES = 16
TW = T // 2
HALF = T // 2


def _sum_pack_body(comps_ref, out_ref):
    c = comps_ref[...]
    s = c[:, 0, :] + c[:, 1, :] + c[:, 2, :] + c[:, 3, :]
    r = lax.bitcast_convert_type(s, jnp.uint32) + jnp.uint32(0x8000)
    ra = r[:, :HALF]
    rb = r[:, HALF:]
    word = (ra >> jnp.uint32(16)) | (rb & jnp.uint32(0xFFFF0000))
    out_ref[...] = lax.bitcast_convert_type(word, jnp.int32)


def _sum_pack(comps):
    bn = 1000
    grid = N // bn
    return pl.pallas_call(
        _sum_pack_body,
        grid=(grid,),
        in_specs=[pl.BlockSpec((bn, 4, T), lambda i: (i, 0, 0))],
        out_specs=pl.BlockSpec((bn, TW), lambda i: (i, 0)),
        out_shape=jax.ShapeDtypeStruct((N, TW), jnp.int32),
    )(comps)


def _dense_body(off_ref, tr_ref, amp_ref, ph_ref, per_ref, tv_ref, out_ref):
    tv = tv_ref[...]
    per = per_ref[...]
    amp = amp_ref[...]
    ph = ph_ref[...]
    off = off_ref[...]
    tr = tr_ref[...]
    ang = (2.0 * np.pi) * tv / per.reshape(5, 1)
    sinb = jnp.sin(ang)
    cosb = jnp.cos(ang)
    base = off + tr * tv
    out_ref[...] = (
        base
        + jnp.dot(amp * jnp.cos(ph), sinb, preferred_element_type=jnp.float32)
        + jnp.dot(amp * jnp.sin(ph), cosb, preferred_element_type=jnp.float32)
    )


def _dense_signals(off, tr, amp, ph, per, tv):
    bn = 1000
    grid = N // bn
    return pl.pallas_call(
        _dense_body,
        grid=(grid,),
        in_specs=[
            pl.BlockSpec((bn, 1), lambda i: (i, 0)),
            pl.BlockSpec((bn, 1), lambda i: (i, 0)),
            pl.BlockSpec((bn, 5), lambda i: (i, 0)),
            pl.BlockSpec((bn, 5), lambda i: (i, 0)),
            pl.BlockSpec((1, 5), lambda i: (0, 0)),
            pl.BlockSpec((1, T), lambda i: (0, 0)),
        ],
        out_specs=pl.BlockSpec((bn, T), lambda i: (i, 0)),
        out_shape=jax.ShapeDtypeStruct((N, T), jnp.float32),
    )(off, tr, amp, ph, per, tv)


def _sc_body(s_hbm, dense_hbm, idx_hbm, w_hbm, out_hbm,
             idx_v, w_v, rows_a, rows_b, acc_a, acc_b,
             sem_a, sem_b, dsem_a, dsem_b, osem_a, osem_b):
    nc = 2
    wid = lax.axis_index("s") * nc + lax.axis_index("c")
    base = wid * SPT
    last = wid == NTILES - 1

    @pl.when(last)
    def _():
        pltpu.sync_copy(idx_hbm.at[pl.ds(base * KP, NLAST * KP)],
                        idx_v.at[pl.ds(0, NLAST * KP)])
        pltpu.sync_copy(w_hbm.at[pl.ds(base * KP, NLAST * KP)],
                        w_v.at[pl.ds(0, NLAST * KP)])

    @pl.when(jnp.logical_not(last))
    def _():
        pltpu.sync_copy(idx_hbm.at[pl.ds(base * KP, SPT * KP)], idx_v)
        pltpu.sync_copy(w_hbm.at[pl.ds(base * KP, SPT * KP)], w_v)

    nch = jnp.where(last, NCH_LAST, NCH)

    gdn = lax.GatherDimensionNumbers(
        offset_dims=(), collapsed_slice_dims=(0,), start_index_map=(0,))

    def splat(vec, k):
        idxs = jnp.full((LANES,), k, jnp.int32)
        return lax.gather(vec, idxs[:, None], dimension_numbers=gdn,
                          slice_sizes=(1,),
                          mode=lax.GatherScatterMode.PROMISE_IN_BOUNDS)

    def gather_h(c, rows_ref, sem):
        return pltpu.make_async_copy(
            s_hbm.at[idx_v.at[pl.ds(c * (CH * KP), CH * KP)]], rows_ref, sem)

    def dense_h(c, acc_ref, sem):
        return pltpu.make_async_copy(
            dense_hbm.at[pl.ds(base + c * CH, CH)], acc_ref, sem)

    def out_h(c, acc_ref, sem):
        return pltpu.make_async_copy(
            acc_ref, out_hbm.at[pl.ds(base + c * CH, CH)], sem)

    def compute(c, rows_ref, acc_ref, sem, dsem, osem):
        dense_h(c, acc_ref, dsem).wait()
        gather_h(c, rows_ref, sem).wait()

        def station(s, carry):
            w_vec = w_v[pl.ds((c * CH + s) * KP, KP)]
            accs = [None] * (TW // LANES)
            for k in range(KP):
                wk = plsc.bitcast(splat(w_vec, k), jnp.bfloat16)
                r = s * KP + k
                for cw in range(TW // LANES):
                    rb = plsc.bitcast(rows_ref[r, pl.ds(cw * LANES, LANES)],
                                      jnp.bfloat16)
                    p = wk * rb
                    accs[cw] = p if k == 0 else accs[cw] + p
            for cw in range(TW // LANES):
                lo, hi = plsc.unpack(accs[cw], format=plsc.PackFormat.INTERLEAVED)
                plsc.addupdate(acc_ref.at[s, pl.ds(cw * LANES, LANES)], lo)
                plsc.addupdate(acc_ref.at[s, pl.ds(HALF + cw * LANES, LANES)], hi)
            return carry

        lax.fori_loop(0, CH, station, 0)
        out_h(c, acc_ref, osem).start()

    gather_h(0, rows_a, sem_a).start()
    dense_h(0, acc_a, dsem_a).start()
    npair = nch // 2

    def pair(i, carry):
        c0 = i * 2
        gather_h(c0 + 1, rows_b, sem_b).start()
        compute(c0, rows_a, acc_a, sem_a, dsem_a, osem_a)

        @pl.when(i < npair - 1)
        def _():
            gather_h(c0 + 2, rows_a, sem_a).start()

        @pl.when(i > 0)
        def _():
            out_h(0, acc_b, osem_b).wait()
        dense_h(c0 + 1, acc_b, dsem_b).start()
        compute(c0 + 1, rows_b, acc_b, sem_b, dsem_b, osem_b)

        @pl.when(i < npair - 1)
        def _():
            out_h(0, acc_a, osem_a).wait()
            dense_h(c0 + 2, acc_a, dsem_a).start()

        return carry

    lax.fori_loop(0, npair, pair, 0)
    out_h(0, acc_a, osem_a).wait()
    out_h(0, acc_b, osem_b).wait()


def _sc_gather(s_tab, dense, idx_flat, w_flat):
    mesh = plsc.VectorSubcoreMesh(core_axis_name="c", subcore_axis_name="s")
    return pl.kernel(
        _sc_body,
        mesh=mesh,
        compiler_params=pltpu.CompilerParams(needs_layout_passes=False),
        out_type=jax.ShapeDtypeStruct((N, T), jnp.float32),
        scratch_types=[
            pltpu.VMEM((SPT * KP,), jnp.int32),
            pltpu.VMEM((SPT * KP,), jnp.int32),
            pltpu.VMEM((CH * KP, TW), jnp.int32),
            pltpu.VMEM((CH * KP, TW), jnp.int32),
            pltpu.VMEM((CH, T), jnp.float32),
            pltpu.VMEM((CH, T), jnp.float32),
            pltpu.SemaphoreType.DMA,
            pltpu.SemaphoreType.DMA,
            pltpu.SemaphoreType.DMA,
            pltpu.SemaphoreType.DMA,
            pltpu.SemaphoreType.DMA,
            pltpu.SemaphoreType.DMA,
        ],
    )(s_tab, dense, idx_flat, w_flat)


@jax.jit
def kernel(time_vector, constant_offset, linear_trend, emd_seasonal_components,
           residual_amplitudes, residual_phases, residual_periods,
           emd_spatial_weights, local_spatial_weights,
           neighbor_indices, neighbor_weights, local_weights):
    f32 = jnp.float32
    amp_p = residual_amplitudes.astype(f32)
    ph_p = residual_phases.astype(f32)
    per_p = residual_periods.astype(f32).reshape(1, 5)
    tv = time_vector.astype(f32).reshape(1, T)
    off = constant_offset.astype(f32).reshape(N, 1)
    tr = linear_trend.astype(f32).reshape(N, 1)

    we = emd_spatial_weights.astype(f32)
    wl = local_spatial_weights.astype(f32)
    cw = wl[:, None] * local_weights.astype(f32) + we[:, None] * neighbor_weights.astype(f32)
    w16 = jnp.concatenate([cw, (1.0 - we - wl)[:, None]], axis=1)
    wbits = lax.bitcast_convert_type(
        w16.astype(jnp.bfloat16), jnp.uint16).astype(jnp.int32)
    w_flat = (wbits | (wbits << 16)).reshape(-1)

    idx_flat = jnp.concatenate(
        [neighbor_indices.astype(jnp.int32),
         jnp.arange(N, dtype=jnp.int32)[:, None]], axis=1).reshape(-1)

    s_packed = _sum_pack(emd_seasonal_components.astype(f32))
    dense = _dense_signals(off, tr, amp_p, ph_p, per_p, tv)
    return _sc_gather(s_packed, dense, idx_flat, w_flat)


# speedup vs baseline: 5.8928x; 1.0025x over previous
import functools
import jax
import jax.numpy as jnp
import numpy as np
from jax import lax
from jax.experimental import pallas as pl
from jax.experimental.pallas import tpu as pltpu
from jax.experimental.pallas import tpu_sc as plsc

N = 10000
K = 15
T = 512
KP = 16
NTILES = 32
SPT = 320
NLAST = N - (NTILES - 1) * SPT
CH = 8
NCH = SPT // CH
NCH_LAST = NLAST // CH
LANES = 16
TW = T // 2
HALF = T // 2


def _sum_pack_body(comps_ref, out_ref):
    c = comps_ref[...]
    s = c[:, 0, :] + c[:, 1, :] + c[:, 2, :] + c[:, 3, :]
    r = lax.bitcast_convert_type(s, jnp.uint32) + jnp.uint32(0x8000)
    ra = r[:, :HALF]
    rb = r[:, HALF:]
    word = (ra >> jnp.uint32(16)) | (rb & jnp.uint32(0xFFFF0000))
    out_ref[...] = lax.bitcast_convert_type(word, jnp.int32)


def _sum_pack(comps):
    bn = 1000
    grid = N // bn
    return pl.pallas_call(
        _sum_pack_body,
        grid=(grid,),
        in_specs=[pl.BlockSpec((bn, 4, T), lambda i: (i, 0, 0))],
        out_specs=pl.BlockSpec((bn, TW), lambda i: (i, 0)),
        out_shape=jax.ShapeDtypeStruct((N, TW), jnp.int32),
    )(comps)


def _dense_body(off_ref, tr_ref, amp_ref, ph_ref, per_ref, tv_ref, out_ref):
    tv = tv_ref[...]
    per = per_ref[...]
    amp = amp_ref[...]
    ph = ph_ref[...]
    off = off_ref[...]
    tr = tr_ref[...]
    ang = (2.0 * np.pi) * tv / per.reshape(5, 1)
    sinb = jnp.sin(ang)
    cosb = jnp.cos(ang)
    base = off + tr * tv
    out_ref[...] = (
        base
        + jnp.dot(amp * jnp.cos(ph), sinb, preferred_element_type=jnp.float32)
        + jnp.dot(amp * jnp.sin(ph), cosb, preferred_element_type=jnp.float32)
    )


def _dense_signals(off, tr, amp, ph, per, tv):
    bn = 1000
    grid = N // bn
    return pl.pallas_call(
        _dense_body,
        grid=(grid,),
        in_specs=[
            pl.BlockSpec((bn, 1), lambda i: (i, 0)),
            pl.BlockSpec((bn, 1), lambda i: (i, 0)),
            pl.BlockSpec((bn, 5), lambda i: (i, 0)),
            pl.BlockSpec((bn, 5), lambda i: (i, 0)),
            pl.BlockSpec((1, 5), lambda i: (0, 0)),
            pl.BlockSpec((1, T), lambda i: (0, 0)),
        ],
        out_specs=pl.BlockSpec((bn, T), lambda i: (i, 0)),
        out_shape=jax.ShapeDtypeStruct((N, T), jnp.float32),
    )(off, tr, amp, ph, per, tv)


def _sc_body(s_hbm, dense_hbm, idx_hbm, w_hbm, out_hbm,
             idx_v, w_v, rows_a, rows_b, acc_a, acc_b,
             sem_a, sem_b, dsem_a, dsem_b, osem_a, osem_b):
    nc = 2
    wid = lax.axis_index("s") * nc + lax.axis_index("c")
    base = wid * SPT
    last = wid == NTILES - 1

    @pl.when(last)
    def _():
        pltpu.sync_copy(idx_hbm.at[pl.ds(base * KP, NLAST * KP)],
                        idx_v.at[pl.ds(0, NLAST * KP)])
        pltpu.sync_copy(w_hbm.at[pl.ds(base, NLAST)],
                        w_v.at[pl.ds(0, NLAST)])

    @pl.when(jnp.logical_not(last))
    def _():
        pltpu.sync_copy(idx_hbm.at[pl.ds(base * KP, SPT * KP)], idx_v)
        pltpu.sync_copy(w_hbm.at[pl.ds(base, SPT)], w_v)

    nch = jnp.where(last, NCH_LAST, NCH)

    gdn = lax.GatherDimensionNumbers(
        offset_dims=(), collapsed_slice_dims=(0,), start_index_map=(0,))

    def splat(vec, k):
        idxs = jnp.full((LANES,), k, jnp.int32)
        return lax.gather(vec, idxs[:, None], dimension_numbers=gdn,
                          slice_sizes=(1,),
                          mode=lax.GatherScatterMode.PROMISE_IN_BOUNDS)

    def gather_h(c, rows_ref, sem):
        return pltpu.make_async_copy(
            s_hbm.at[idx_v.at[pl.ds(c * (CH * KP), CH * KP)]], rows_ref, sem)

    def dense_h(c, acc_ref, sem):
        return pltpu.make_async_copy(
            dense_hbm.at[pl.ds(base + c * CH, CH)], acc_ref, sem)

    def out_h(c, acc_ref, sem):
        return pltpu.make_async_copy(
            acc_ref, out_hbm.at[pl.ds(base + c * CH, CH)], sem)

    def compute(c, rows_ref, acc_ref, sem, dsem, osem):
        dense_h(c, acc_ref, dsem).wait()
        gather_h(c, rows_ref, sem).wait()

        def station(s, carry):
            w_vec = w_v[c * CH + s, :]
            accs = [None] * (TW // LANES)
            for k in range(KP):
                wk = plsc.bitcast(splat(w_vec, k), jnp.bfloat16)
                r = s * KP + k
                for cw in range(TW // LANES):
                    rb = plsc.bitcast(rows_ref[r, pl.ds(cw * LANES, LANES)],
                                      jnp.bfloat16)
                    p = wk * rb
                    accs[cw] = p if k == 0 else accs[cw] + p
            for cw in range(TW // LANES):
                lo, hi = plsc.unpack(accs[cw], format=plsc.PackFormat.INTERLEAVED)
                plsc.addupdate(acc_ref.at[s, pl.ds(cw * LANES, LANES)], lo)
                plsc.addupdate(acc_ref.at[s, pl.ds(HALF + cw * LANES, LANES)], hi)
            return carry

        lax.fori_loop(0, CH, station, 0)
        out_h(c, acc_ref, osem).start()

    gather_h(0, rows_a, sem_a).start()
    dense_h(0, acc_a, dsem_a).start()
    npair = nch // 2

    def pair(i, carry):
        c0 = i * 2
        gather_h(c0 + 1, rows_b, sem_b).start()
        compute(c0, rows_a, acc_a, sem_a, dsem_a, osem_a)

        @pl.when(i < npair - 1)
        def _():
            gather_h(c0 + 2, rows_a, sem_a).start()

        @pl.when(i > 0)
        def _():
            out_h(0, acc_b, osem_b).wait()
        dense_h(c0 + 1, acc_b, dsem_b).start()
        compute(c0 + 1, rows_b, acc_b, sem_b, dsem_b, osem_b)

        @pl.when(i < npair - 1)
        def _():
            out_h(0, acc_a, osem_a).wait()
            dense_h(c0 + 2, acc_a, dsem_a).start()

        return carry

    lax.fori_loop(0, npair, pair, 0)
    out_h(0, acc_a, osem_a).wait()
    out_h(0, acc_b, osem_b).wait()


def _sc_gather(s_tab, dense, idx_flat, w_flat):
    mesh = plsc.VectorSubcoreMesh(core_axis_name="c", subcore_axis_name="s")
    return pl.kernel(
        _sc_body,
        mesh=mesh,
        compiler_params=pltpu.CompilerParams(needs_layout_passes=False),
        out_type=jax.ShapeDtypeStruct((N, T), jnp.float32),
        scratch_types=[
            pltpu.VMEM((SPT * KP,), jnp.int32),
            pltpu.VMEM((SPT, KP), jnp.int32),
            pltpu.VMEM((CH * KP, TW), jnp.int32),
            pltpu.VMEM((CH * KP, TW), jnp.int32),
            pltpu.VMEM((CH, T), jnp.float32),
            pltpu.VMEM((CH, T), jnp.float32),
            pltpu.SemaphoreType.DMA,
            pltpu.SemaphoreType.DMA,
            pltpu.SemaphoreType.DMA,
            pltpu.SemaphoreType.DMA,
            pltpu.SemaphoreType.DMA,
            pltpu.SemaphoreType.DMA,
        ],
    )(s_tab, dense, idx_flat, w_flat)


@jax.jit
def kernel(time_vector, constant_offset, linear_trend, emd_seasonal_components,
           residual_amplitudes, residual_phases, residual_periods,
           emd_spatial_weights, local_spatial_weights,
           neighbor_indices, neighbor_weights, local_weights):
    f32 = jnp.float32
    amp_p = residual_amplitudes.astype(f32)
    ph_p = residual_phases.astype(f32)
    per_p = residual_periods.astype(f32).reshape(1, 5)
    tv = time_vector.astype(f32).reshape(1, T)
    off = constant_offset.astype(f32).reshape(N, 1)
    tr = linear_trend.astype(f32).reshape(N, 1)

    we = emd_spatial_weights.astype(f32)
    wl = local_spatial_weights.astype(f32)
    cw = wl[:, None] * local_weights.astype(f32) + we[:, None] * neighbor_weights.astype(f32)
    w16 = jnp.concatenate([cw, (1.0 - we - wl)[:, None]], axis=1)
    wbits = lax.bitcast_convert_type(
        w16.astype(jnp.bfloat16), jnp.uint16).astype(jnp.int32)
    w_tab = wbits | (wbits << 16)

    idx_flat = jnp.concatenate(
        [neighbor_indices.astype(jnp.int32),
         jnp.arange(N, dtype=jnp.int32)[:, None]], axis=1).reshape(-1)

    s_packed = _sum_pack(emd_seasonal_components.astype(f32))
    dense = _dense_signals(off, tr, amp_p, ph_p, per_p, tv)
    return _sc_gather(s_packed, dense, idx_flat, w_tab)
